# baseline (speedup 1.0000x reference)
import functools

import jax
import jax.numpy as jnp
from jax import lax
from jax.experimental import pallas as pl
from jax.experimental.pallas import tpu as pltpu
from jax.experimental.pallas import tpu_sc as plsc

N = 320000
D = 128
B = 64
NUM_CORES = 2
NUM_SUBCORES = 16
NUM_WORKERS = NUM_CORES * NUM_SUBCORES
ROWS_PER_WORKER = N // NUM_WORKERS
LANES = 16
GROUPS_PER_WORKER = ROWS_PER_WORKER // LANES
GROUPS_PER_CHUNK = 5
CHUNK_ROWS = GROUPS_PER_CHUNK * LANES
CHUNK_WORDS = CHUNK_ROWS * D
NUM_CHUNKS = GROUPS_PER_WORKER // GROUPS_PER_CHUNK


def _sc_partials(pred2d, tgt2d, bidx, flag):
    mesh = plsc.VectorSubcoreMesh(core_axis_name="c", subcore_axis_name="s")

    @functools.partial(
        pl.kernel,
        mesh=mesh,
        compiler_params=pltpu.CompilerParams(needs_layout_passes=False),
        out_type=(
            jax.ShapeDtypeStruct((NUM_WORKERS, B, LANES), jnp.float32),
            jax.ShapeDtypeStruct((NUM_WORKERS, B, LANES), jnp.float32),
        ),
        scratch_types=[
            pltpu.VMEM((CHUNK_ROWS, D), jnp.float32),
            pltpu.VMEM((CHUNK_ROWS, D), jnp.float32),
            pltpu.VMEM((ROWS_PER_WORKER,), jnp.int32),
            pltpu.VMEM((ROWS_PER_WORKER,), jnp.float32),
            pltpu.VMEM((B, LANES), jnp.float32),
            pltpu.VMEM((B, LANES), jnp.float32),
        ],
    )
    def k(pred_hbm, tgt_hbm, bidx_hbm, flag_hbm, num_out, cnt_out,
          pbuf, tbuf, bbuf, fbuf, tab_num, tab_cnt):
        wid = lax.axis_index("c") * NUM_SUBCORES + lax.axis_index("s")
        row0 = wid * ROWS_PER_WORKER
        iota = lax.iota(jnp.int32, LANES)
        zeros = jnp.zeros((LANES,), jnp.float32)

        pltpu.sync_copy(bidx_hbm.at[pl.ds(row0, ROWS_PER_WORKER)], bbuf)
        pltpu.sync_copy(flag_hbm.at[pl.ds(row0, ROWS_PER_WORKER)], fbuf)

        def zero_body(r, _):
            tab_num[r] = zeros
            tab_cnt[r] = zeros
            return _
        lax.fori_loop(0, B, zero_body, 0)

        def chunk_body(c, _):
            rbase = row0 + c * CHUNK_ROWS
            pltpu.sync_copy(pred_hbm.at[pl.ds(rbase, CHUNK_ROWS)], pbuf)
            pltpu.sync_copy(tgt_hbm.at[pl.ds(rbase, CHUNK_ROWS)], tbuf)

            def group_body(gi, _):
                g = c * GROUPS_PER_CHUNK + gi
                bidx16 = bbuf[pl.ds(g * LANES, LANES)]
                flag16 = fbuf[pl.ds(g * LANES, LANES)]
                rowvec = iota + gi * LANES

                def d_body(d, mse):
                    dvec = jnp.full((LANES,), d, jnp.int32)
                    pv = plsc.load_gather(pbuf, [rowvec, dvec])
                    tv = plsc.load_gather(tbuf, [rowvec, dvec])
                    df = pv - tv
                    return mse + df * df
                mse = lax.fori_loop(0, D, d_body, zeros)

                plsc.addupdate_scatter(tab_num, [bidx16, iota], mse * flag16)
                plsc.addupdate_scatter(tab_cnt, [bidx16, iota], flag16)
                return _
            lax.fori_loop(0, GROUPS_PER_CHUNK, group_body, 0)
            return _
        lax.fori_loop(0, NUM_CHUNKS, chunk_body, 0)

        pltpu.sync_copy(tab_num, num_out.at[wid])
        pltpu.sync_copy(tab_cnt, cnt_out.at[wid])

    return k(pred2d, tgt2d, bidx, flag)


def _tc_combine_body(num_ref, cnt_ref, out_ref):
    num = jnp.sum(jnp.sum(num_ref[...], axis=2), axis=0)
    cnt = jnp.sum(jnp.sum(cnt_ref[...], axis=2), axis=0)
    loss = num / jnp.maximum(cnt, 1.0)
    out_ref[...] = jnp.mean(loss).reshape(1, 1)


@jax.jit
def kernel(pred, tgt, t, gen_flag, batch_idx, sigmas):
    del t, sigmas
    bidx = batch_idx.astype(jnp.int32)
    flag = gen_flag.astype(jnp.float32)

    num_parts, cnt_parts = _sc_partials(pred, tgt, bidx, flag)

    out = pl.pallas_call(
        _tc_combine_body,
        out_shape=jax.ShapeDtypeStruct((1, 1), jnp.float32),
    )(num_parts, cnt_parts)
    return out[0, 0]


# speedup vs baseline: 1.2742x; 1.2742x over previous
import functools

import jax
import jax.numpy as jnp
from jax import lax
from jax.experimental import pallas as pl
from jax.experimental.pallas import tpu as pltpu
from jax.experimental.pallas import tpu_sc as plsc

N = 320000
D = 128
B = 64
NUM_CORES = 2
NUM_SUBCORES = 16
NUM_WORKERS = NUM_CORES * NUM_SUBCORES
ROWS_PER_WORKER = N // NUM_WORKERS
LANES = 16
GROUPS_PER_WORKER = ROWS_PER_WORKER // LANES
GROUPS_PER_CHUNK = 5
CHUNK_ROWS = GROUPS_PER_CHUNK * LANES
CHUNK_WORDS = CHUNK_ROWS * D
NUM_CHUNKS = GROUPS_PER_WORKER // GROUPS_PER_CHUNK
D_UNROLL = 32
D_OUTER = D // D_UNROLL


def _sc_partials(pred_flat, tgt_flat, bidx, flag):
    mesh = plsc.VectorSubcoreMesh(core_axis_name="c", subcore_axis_name="s")

    @functools.partial(
        pl.kernel,
        mesh=mesh,
        compiler_params=pltpu.CompilerParams(needs_layout_passes=False),
        out_type=(
            jax.ShapeDtypeStruct((NUM_WORKERS, B, LANES), jnp.float32),
            jax.ShapeDtypeStruct((NUM_WORKERS, B, LANES), jnp.float32),
        ),
        scratch_types=[
            pltpu.VMEM((CHUNK_WORDS,), jnp.float32),
            pltpu.VMEM((CHUNK_WORDS,), jnp.float32),
            pltpu.VMEM((CHUNK_WORDS,), jnp.float32),
            pltpu.VMEM((CHUNK_WORDS,), jnp.float32),
            pltpu.VMEM((ROWS_PER_WORKER,), jnp.int32),
            pltpu.VMEM((ROWS_PER_WORKER,), jnp.float32),
            pltpu.VMEM((B, LANES), jnp.float32),
            pltpu.VMEM((B, LANES), jnp.float32),
            pltpu.SemaphoreType.DMA,
            pltpu.SemaphoreType.DMA,
            pltpu.SemaphoreType.DMA,
            pltpu.SemaphoreType.DMA,
        ],
    )
    def k(pred_hbm, tgt_hbm, bidx_hbm, flag_hbm, num_out, cnt_out,
          pbuf0, pbuf1, tbuf0, tbuf1, bbuf, fbuf, tab_num, tab_cnt,
          semp0, semp1, semt0, semt1):
        wid = lax.axis_index("c") * NUM_SUBCORES + lax.axis_index("s")
        row0 = wid * ROWS_PER_WORKER
        iota = lax.iota(jnp.int32, LANES)
        zeros = jnp.zeros((LANES,), jnp.float32)

        pltpu.sync_copy(bidx_hbm.at[pl.ds(row0, ROWS_PER_WORKER)], bbuf)
        pltpu.sync_copy(flag_hbm.at[pl.ds(row0, ROWS_PER_WORKER)], fbuf)

        def zero_body(r, _):
            tab_num[r] = zeros
            tab_cnt[r] = zeros
            return _
        lax.fori_loop(0, B, zero_body, 0)

        def start(c, pb, tb, semp, semt):
            base = (row0 + c * CHUNK_ROWS) * D
            pltpu.async_copy(pred_hbm.at[pl.ds(base, CHUNK_WORDS)], pb, semp)
            pltpu.async_copy(tgt_hbm.at[pl.ds(base, CHUNK_WORDS)], tb, semt)

        def drain(pb, tb, semp, semt):
            pltpu.make_async_copy(
                pred_hbm.at[pl.ds(0, CHUNK_WORDS)], pb, semp).wait()
            pltpu.make_async_copy(
                tgt_hbm.at[pl.ds(0, CHUNK_WORDS)], tb, semt).wait()

        def compute(c, pb, tb):
            def group_body(gi, _):
                g = c * GROUPS_PER_CHUNK + gi
                bidx16 = bbuf[pl.ds(g * LANES, LANES)]
                flag16 = fbuf[pl.ds(g * LANES, LANES)]
                basevec = iota * D + gi * (LANES * D)

                def d_body(dd, accs):
                    a0, a1, a2, a3 = accs
                    base_dd = basevec + dd * D_UNROLL
                    for j in range(0, D_UNROLL, 4):
                        for q in range(4):
                            idx = base_dd + (j + q)
                            pv = plsc.load_gather(pb, [idx])
                            tv = plsc.load_gather(tb, [idx])
                            df = pv - tv
                            if q == 0:
                                a0 = a0 + df * df
                            elif q == 1:
                                a1 = a1 + df * df
                            elif q == 2:
                                a2 = a2 + df * df
                            else:
                                a3 = a3 + df * df
                    return (a0, a1, a2, a3)
                a0, a1, a2, a3 = lax.fori_loop(
                    0, D_OUTER, d_body, (zeros, zeros, zeros, zeros))
                mse = (a0 + a1) + (a2 + a3)

                plsc.addupdate_scatter(tab_num, [bidx16, iota], mse * flag16)
                plsc.addupdate_scatter(tab_cnt, [bidx16, iota], flag16)
                return _
            lax.fori_loop(0, GROUPS_PER_CHUNK, group_body, 0)

        start(0, pbuf0, tbuf0, semp0, semt0)

        def pair_body(p, _):
            c0 = 2 * p
            start(c0 + 1, pbuf1, tbuf1, semp1, semt1)
            drain(pbuf0, tbuf0, semp0, semt0)
            compute(c0, pbuf0, tbuf0)
            start(c0 + 2, pbuf0, tbuf0, semp0, semt0)
            drain(pbuf1, tbuf1, semp1, semt1)
            compute(c0 + 1, pbuf1, tbuf1)
            return _
        lax.fori_loop(0, (NUM_CHUNKS - 1) // 2, pair_body, 0)

        drain(pbuf0, tbuf0, semp0, semt0)
        compute(NUM_CHUNKS - 1, pbuf0, tbuf0)

        pltpu.sync_copy(tab_num, num_out.at[wid])
        pltpu.sync_copy(tab_cnt, cnt_out.at[wid])

    return k(pred_flat, tgt_flat, bidx, flag)


def _tc_combine_body(num_ref, cnt_ref, out_ref):
    num = jnp.sum(jnp.sum(num_ref[...], axis=2), axis=0)
    cnt = jnp.sum(jnp.sum(cnt_ref[...], axis=2), axis=0)
    loss = num / jnp.maximum(cnt, 1.0)
    out_ref[...] = jnp.mean(loss).reshape(1, 1)


@jax.jit
def kernel(pred, tgt, t, gen_flag, batch_idx, sigmas):
    del t, sigmas
    bidx = batch_idx.astype(jnp.int32)
    flag = gen_flag.astype(jnp.float32)

    num_parts, cnt_parts = _sc_partials(
        pred.reshape(-1), tgt.reshape(-1), bidx, flag)

    out = pl.pallas_call(
        _tc_combine_body,
        out_shape=jax.ShapeDtypeStruct((1, 1), jnp.float32),
    )(num_parts, cnt_parts)
    return out[0, 0]


# speedup vs baseline: 8.7603x; 6.8753x over previous
import functools

import jax
import jax.numpy as jnp
from jax import lax
from jax.experimental import pallas as pl
from jax.experimental.pallas import tpu as pltpu
from jax.experimental.pallas import tpu_sc as plsc

N = 320000
D = 128
B = 64
NUM_CORES = 2
NUM_SUBCORES = 16
NUM_WORKERS = NUM_CORES * NUM_SUBCORES
ROWS_PER_WORKER = N // NUM_WORKERS
LANES = 16
GROUPS_PER_WORKER = ROWS_PER_WORKER // LANES
GROUPS_PER_CHUNK = 5
CHUNK_ROWS = GROUPS_PER_CHUNK * LANES
CHUNK_WORDS = CHUNK_ROWS * D
NUM_CHUNKS = GROUPS_PER_WORKER // GROUPS_PER_CHUNK
D_UNROLL = 32
D_OUTER = D // D_UNROLL


def _sc_partials(pred_flat, tgt_flat, bidx, flag):
    mesh = plsc.VectorSubcoreMesh(core_axis_name="c", subcore_axis_name="s")

    @functools.partial(
        pl.kernel,
        mesh=mesh,
        compiler_params=pltpu.CompilerParams(needs_layout_passes=False),
        out_type=(
            jax.ShapeDtypeStruct((NUM_WORKERS, B, LANES), jnp.float32),
            jax.ShapeDtypeStruct((NUM_WORKERS, B, LANES), jnp.float32),
        ),
        scratch_types=[
            pltpu.VMEM((CHUNK_WORDS,), jnp.float32),
            pltpu.VMEM((CHUNK_WORDS,), jnp.float32),
            pltpu.VMEM((CHUNK_WORDS,), jnp.float32),
            pltpu.VMEM((CHUNK_WORDS,), jnp.float32),
            pltpu.VMEM((ROWS_PER_WORKER,), jnp.int32),
            pltpu.VMEM((ROWS_PER_WORKER,), jnp.float32),
            pltpu.VMEM((B, LANES), jnp.float32),
            pltpu.VMEM((B, LANES), jnp.float32),
            pltpu.SemaphoreType.DMA,
            pltpu.SemaphoreType.DMA,
            pltpu.SemaphoreType.DMA,
            pltpu.SemaphoreType.DMA,
        ],
    )
    def k(pred_hbm, tgt_hbm, bidx_hbm, flag_hbm, num_out, cnt_out,
          pbuf0, pbuf1, tbuf0, tbuf1, bbuf, fbuf, tab_num, tab_cnt,
          semp0, semp1, semt0, semt1):
        wid = lax.axis_index("c") * NUM_SUBCORES + lax.axis_index("s")
        row0 = wid * ROWS_PER_WORKER
        iota = lax.iota(jnp.int32, LANES)
        zeros = jnp.zeros((LANES,), jnp.float32)

        pltpu.sync_copy(bidx_hbm.at[pl.ds(row0, ROWS_PER_WORKER)], bbuf)
        pltpu.sync_copy(flag_hbm.at[pl.ds(row0, ROWS_PER_WORKER)], fbuf)

        def zero_body(r, _):
            tab_num[r] = zeros
            tab_cnt[r] = zeros
            return _
        lax.fori_loop(0, B, zero_body, 0)

        def start(c, pb, tb, semp, semt):
            base = (row0 + c * CHUNK_ROWS) * D
            pltpu.async_copy(pred_hbm.at[pl.ds(base, CHUNK_WORDS)], pb, semp)
            pltpu.async_copy(tgt_hbm.at[pl.ds(base, CHUNK_WORDS)], tb, semt)

        def drain(pb, tb, semp, semt):
            pltpu.make_async_copy(
                pred_hbm.at[pl.ds(0, CHUNK_WORDS)], pb, semp).wait()
            pltpu.make_async_copy(
                tgt_hbm.at[pl.ds(0, CHUNK_WORDS)], tb, semt).wait()

        def compute(c, pb, tb):
            def group_body(gi, _):
                g = c * GROUPS_PER_CHUNK + gi
                bidx16 = bbuf[pl.ds(g * LANES, LANES)]
                flag16 = fbuf[pl.ds(g * LANES, LANES)]
                basevec = iota * D + gi * (LANES * D)

                def d_body(dd, accs):
                    a0, a1, a2, a3 = accs
                    rot = iota + dd * D_UNROLL
                    for j in range(0, D_UNROLL, 4):
                        for q in range(4):
                            col = (rot + j + q) & (D - 1)
                            idx = basevec + col
                            pv = plsc.load_gather(pb, [idx])
                            tv = plsc.load_gather(tb, [idx])
                            df = pv - tv
                            if q == 0:
                                a0 = a0 + df * df
                            elif q == 1:
                                a1 = a1 + df * df
                            elif q == 2:
                                a2 = a2 + df * df
                            else:
                                a3 = a3 + df * df
                    return (a0, a1, a2, a3)
                a0, a1, a2, a3 = lax.fori_loop(
                    0, D_OUTER, d_body, (zeros, zeros, zeros, zeros))
                mse = (a0 + a1) + (a2 + a3)

                plsc.addupdate_scatter(tab_num, [bidx16, iota], mse * flag16)
                plsc.addupdate_scatter(tab_cnt, [bidx16, iota], flag16)
                return _
            lax.fori_loop(0, GROUPS_PER_CHUNK, group_body, 0)

        start(0, pbuf0, tbuf0, semp0, semt0)

        def pair_body(p, _):
            c0 = 2 * p
            start(c0 + 1, pbuf1, tbuf1, semp1, semt1)
            drain(pbuf0, tbuf0, semp0, semt0)
            compute(c0, pbuf0, tbuf0)
            start(c0 + 2, pbuf0, tbuf0, semp0, semt0)
            drain(pbuf1, tbuf1, semp1, semt1)
            compute(c0 + 1, pbuf1, tbuf1)
            return _
        lax.fori_loop(0, (NUM_CHUNKS - 1) // 2, pair_body, 0)

        drain(pbuf0, tbuf0, semp0, semt0)
        compute(NUM_CHUNKS - 1, pbuf0, tbuf0)

        pltpu.sync_copy(tab_num, num_out.at[wid])
        pltpu.sync_copy(tab_cnt, cnt_out.at[wid])

    return k(pred_flat, tgt_flat, bidx, flag)


def _tc_combine_body(num_ref, cnt_ref, out_ref):
    num = jnp.sum(jnp.sum(num_ref[...], axis=2), axis=0)
    cnt = jnp.sum(jnp.sum(cnt_ref[...], axis=2), axis=0)
    loss = num / jnp.maximum(cnt, 1.0)
    out_ref[...] = jnp.mean(loss).reshape(1, 1)


@jax.jit
def kernel(pred, tgt, t, gen_flag, batch_idx, sigmas):
    del t, sigmas
    bidx = batch_idx.astype(jnp.int32)
    flag = gen_flag.astype(jnp.float32)

    num_parts, cnt_parts = _sc_partials(
        pred.reshape(-1), tgt.reshape(-1), bidx, flag)

    out = pl.pallas_call(
        _tc_combine_body,
        out_shape=jax.ShapeDtypeStruct((1, 1), jnp.float32),
    )(num_parts, cnt_parts)
    return out[0, 0]
